# baseline (speedup 1.0000x reference)
import functools

import jax
import jax.numpy as jnp
from jax import lax
from jax.experimental import pallas as pl
from jax.experimental.pallas import tpu as pltpu
from jax.experimental.pallas import tpu_sc as plsc

_B = 16384
_D = 64
_H = 256
_V = 100001
_NC = 2
_NS = 16
_NW = _NC * _NS
_BPW = _B // _NW
_CHUNK = 128
_NCH = _BPW // _CHUNK
_NTC = (_V + 127) // 128
_LASTW = _V - (_NTC - 1) * 128
_LINR = _NTC * 64
_ITERS = (_NTC + _NW - 1) // _NW
_BLK = 2048


def _mesh():
    return plsc.VectorSubcoreMesh(core_axis_name="c", subcore_axis_name="s")


def _sc_format(tableT, tail):
    @functools.partial(
        pl.kernel,
        mesh=_mesh(),
        out_type=jax.ShapeDtypeStruct((_LINR, 128), jnp.float32),
        scratch_types=[
            pltpu.VMEM((_D, 128), jnp.float32),
            pltpu.VMEM((_D, 128), jnp.float32),
        ],
        compiler_params=pltpu.CompilerParams(needs_layout_passes=False),
    )
    def k(tT_hbm, tail_hbm, lin_hbm, bin_, bout):
        w = lax.axis_index("s") * _NC + lax.axis_index("c")
        rowv = [lax.iota(jnp.int32, 16) + 16 * cb for cb in range(4)]

        def body(i, carry):
            tc = w + _NW * i

            @pl.when(tc < _NTC - 1)
            def _load_full():
                off = pl.multiple_of(tc * 128, 128)
                pltpu.sync_copy(tT_hbm.at[:, pl.ds(off, 128)], bin_)

            @pl.when(tc == _NTC - 1)
            def _load_part():
                pltpu.sync_copy(tail_hbm, bin_)

            @pl.when(tc < _NTC)
            def _work():
                for j in range(64):
                    for a in range(2):
                        colv = jnp.full((16,), 2 * j + a, jnp.int32)
                        for cb in range(4):
                            vals = plsc.load_gather(bin_, [rowv[cb], colv])
                            bout[j, pl.ds(64 * a + 16 * cb, 16)] = vals
                pltpu.sync_copy(bout, lin_hbm.at[pl.ds(tc * 64, 64), :])

            return carry

        lax.fori_loop(0, _ITERS, body, 0)

    return k(tableT, tail)


def _sc_gather(lin, labels):
    @functools.partial(
        pl.kernel,
        mesh=_mesh(),
        out_type=jax.ShapeDtypeStruct((_B, 128), jnp.float32),
        scratch_types=[
            pltpu.VMEM((_BPW,), jnp.int32),
            pltpu.VMEM((_BPW,), jnp.int32),
            pltpu.VMEM((_BPW, 128), jnp.float32),
            pltpu.SemaphoreType.DMA,
        ],
        compiler_params=pltpu.CompilerParams(needs_layout_passes=False),
    )
    def k(lin_hbm, lab_hbm, out_hbm, lab_v, pidx_v, rows_v, sem):
        w = lax.axis_index("s") * _NC + lax.axis_index("c")
        base = w * _BPW
        pltpu.sync_copy(lab_hbm.at[pl.ds(base, _BPW)], lab_v)
        for t in range(_BPW // 16):
            pidx_v[pl.ds(16 * t, 16)] = lab_v[pl.ds(16 * t, 16)] >> 1
        copies = []
        for j in range(_NCH):
            copies.append(
                pltpu.async_copy(
                    lin_hbm.at[pidx_v.at[pl.ds(j * _CHUNK, _CHUNK)]],
                    rows_v.at[pl.ds(j * _CHUNK, _CHUNK)],
                    sem,
                )
            )
        for c in copies:
            c.wait()
        pltpu.sync_copy(rows_v, out_hbm.at[pl.ds(base, _BPW)])

    return k(lin, labels)


def _mlp_body(x_ref, lab_ref, w1_ref, b1_ref, w2_ref, b2_ref, o_ref):
    x = x_ref[...]
    par = lab_ref[...].reshape(_BLK, 1) & 1
    e = jnp.where(par == 1, x[:, _D:2 * _D], x[:, 0:_D])
    h = jnp.dot(e, w1_ref[...], preferred_element_type=jnp.float32)
    h = h + b1_ref[...]
    h = h * jax.nn.sigmoid(h)
    o = jnp.dot(h, w2_ref[...], preferred_element_type=jnp.float32)
    o_ref[...] = (o + b2_ref[...]).T


def _tc_mlp(emb128, labels3, W1, b1, W2, b2):
    grid = (_B // _BLK,)
    return pl.pallas_call(
        _mlp_body,
        grid=grid,
        in_specs=[
            pl.BlockSpec((_BLK, 128), lambda i: (i, 0)),
            pl.BlockSpec((1, 1, _BLK), lambda i: (i, 0, 0)),
            pl.BlockSpec((_D, _H), lambda i: (0, 0)),
            pl.BlockSpec((1, _H), lambda i: (0, 0)),
            pl.BlockSpec((_H, _D), lambda i: (0, 0)),
            pl.BlockSpec((1, _D), lambda i: (0, 0)),
        ],
        out_specs=pl.BlockSpec((_D, _BLK), lambda i: (0, i)),
        out_shape=jax.ShapeDtypeStruct((_D, _B), jnp.float32),
    )(emb128, labels3, W1, b1, W2, b2)


def kernel(class_labels, table, W1, b1, W2, b2):
    labels = class_labels.astype(jnp.int32)
    tableT = table.T
    tail = jnp.pad(tableT[:, (_NTC - 1) * 128:], ((0, 0), (0, 128 - _LASTW)))
    lin = _sc_format(tableT, tail)
    emb128 = _sc_gather(lin, labels)
    labels3 = labels.reshape(_B // _BLK, 1, _BLK)
    outT = _tc_mlp(emb128, labels3, W1, b1.reshape(1, _H), W2, b2.reshape(1, _D))
    return outT.T


# speedup vs baseline: 1.0169x; 1.0169x over previous
import functools

import jax
import jax.numpy as jnp
from jax import lax
from jax.experimental import pallas as pl
from jax.experimental.pallas import tpu as pltpu
from jax.experimental.pallas import tpu_sc as plsc

_B = 16384
_D = 64
_H = 256
_V = 100001
_NC = 2
_NS = 16
_NW = _NC * _NS
_BPW = _B // _NW
_CHUNK = 128
_NCH = _BPW // _CHUNK
_NTC = (_V + 127) // 128
_LASTW = _V - (_NTC - 1) * 128
_LINR = _NTC * 64
_ITERS = (_NTC + _NW - 1) // _NW
_BLK = 2048


def _mesh():
    return plsc.VectorSubcoreMesh(core_axis_name="c", subcore_axis_name="s")


def _sc_format(tableT, tail):
    @functools.partial(
        pl.kernel,
        mesh=_mesh(),
        out_type=jax.ShapeDtypeStruct((_LINR, 128), jnp.float32),
        scratch_types=[
            pltpu.VMEM((_D, 129), jnp.float32),
            pltpu.VMEM((_D, 128), jnp.float32),
        ],
        compiler_params=pltpu.CompilerParams(needs_layout_passes=False),
    )
    def k(tT_hbm, tail_hbm, lin_hbm, bin_, bout):
        w = lax.axis_index("s") * _NC + lax.axis_index("c")
        rowv = [lax.iota(jnp.int32, 16) + 16 * cb for cb in range(4)]

        def body(i, carry):
            tc = w + _NW * i

            @pl.when(tc < _NTC - 1)
            def _load_full():
                off = pl.multiple_of(tc * 128, 128)
                pltpu.sync_copy(tT_hbm.at[:, pl.ds(off, 128)], bin_.at[:, pl.ds(0, 128)])

            @pl.when(tc == _NTC - 1)
            def _load_part():
                pltpu.sync_copy(tail_hbm, bin_.at[:, pl.ds(0, 128)])

            @pl.when(tc < _NTC)
            def _work():
                def jblk(jb, carry2):
                    for jj in range(8):
                        j = jb * 8 + jj
                        for a in range(2):
                            colv = lax.broadcast(2 * j + a, (16,))
                            for cb in range(4):
                                vals = plsc.load_gather(bin_, [rowv[cb], colv])
                                bout[j, pl.ds(64 * a + 16 * cb, 16)] = vals
                    return carry2

                lax.fori_loop(0, 8, jblk, 0)
                pltpu.sync_copy(bout, lin_hbm.at[pl.ds(tc * 64, 64), :])

            return carry

        lax.fori_loop(0, _ITERS, body, 0)

    return k(tableT, tail)


def _sc_gather(lin, labels):
    @functools.partial(
        pl.kernel,
        mesh=_mesh(),
        out_type=jax.ShapeDtypeStruct((_B, 128), jnp.float32),
        scratch_types=[
            pltpu.VMEM((_BPW,), jnp.int32),
            pltpu.VMEM((_BPW,), jnp.int32),
            pltpu.VMEM((_BPW, 128), jnp.float32),
            pltpu.SemaphoreType.DMA,
        ],
        compiler_params=pltpu.CompilerParams(needs_layout_passes=False),
    )
    def k(lin_hbm, lab_hbm, out_hbm, lab_v, pidx_v, rows_v, sem):
        w = lax.axis_index("s") * _NC + lax.axis_index("c")
        base = w * _BPW
        pltpu.sync_copy(lab_hbm.at[pl.ds(base, _BPW)], lab_v)
        for t in range(_BPW // 16):
            pidx_v[pl.ds(16 * t, 16)] = lab_v[pl.ds(16 * t, 16)] >> 1
        copies = []
        for j in range(_NCH):
            copies.append(
                pltpu.async_copy(
                    lin_hbm.at[pidx_v.at[pl.ds(j * _CHUNK, _CHUNK)]],
                    rows_v.at[pl.ds(j * _CHUNK, _CHUNK)],
                    sem,
                )
            )
        for c in copies:
            c.wait()
        pltpu.sync_copy(rows_v, out_hbm.at[pl.ds(base, _BPW)])

    return k(lin, labels)


def _mlp_body(x_ref, lab_ref, w1_ref, b1_ref, w2_ref, b2_ref, o_ref):
    x = x_ref[...]
    par = lab_ref[...].reshape(_BLK, 1) & 1
    e = jnp.where(par == 1, x[:, _D:2 * _D], x[:, 0:_D])
    h = jnp.dot(e, w1_ref[...], preferred_element_type=jnp.float32)
    h = h + b1_ref[...]
    h = h * jax.nn.sigmoid(h)
    o = jnp.dot(h, w2_ref[...], preferred_element_type=jnp.float32)
    o_ref[...] = (o + b2_ref[...]).T


def _tc_mlp(emb128, labels3, W1, b1, W2, b2):
    grid = (_B // _BLK,)
    return pl.pallas_call(
        _mlp_body,
        grid=grid,
        in_specs=[
            pl.BlockSpec((_BLK, 128), lambda i: (i, 0)),
            pl.BlockSpec((1, 1, _BLK), lambda i: (i, 0, 0)),
            pl.BlockSpec((_D, _H), lambda i: (0, 0)),
            pl.BlockSpec((1, _H), lambda i: (0, 0)),
            pl.BlockSpec((_H, _D), lambda i: (0, 0)),
            pl.BlockSpec((1, _D), lambda i: (0, 0)),
        ],
        out_specs=pl.BlockSpec((_D, _BLK), lambda i: (0, i)),
        out_shape=jax.ShapeDtypeStruct((_D, _B), jnp.float32),
    )(emb128, labels3, W1, b1, W2, b2)


def kernel(class_labels, table, W1, b1, W2, b2):
    labels = class_labels.astype(jnp.int32)
    tableT = table.T
    tail = jnp.pad(tableT[:, (_NTC - 1) * 128:], ((0, 0), (0, 128 - _LASTW)))
    lin = _sc_format(tableT, tail)
    emb128 = _sc_gather(lin, labels)
    labels3 = labels.reshape(_B // _BLK, 1, _BLK)
    outT = _tc_mlp(emb128, labels3, W1, b1.reshape(1, _H), W2, b2.reshape(1, _D))
    return outT.T


# speedup vs baseline: 1.4737x; 1.4493x over previous
import functools

import jax
import jax.numpy as jnp
from jax import lax
from jax.experimental import pallas as pl
from jax.experimental.pallas import tpu as pltpu
from jax.experimental.pallas import tpu_sc as plsc

_B = 16384
_D = 64
_H = 256
_V = 100001
_NC = 2
_NS = 16
_NW = _NC * _NS
_BPW = _B // _NW
_CHUNK = 128
_NCH = _BPW // _CHUNK
_NTC = (_V + 127) // 128
_LASTW = _V - (_NTC - 1) * 128
_LINR = _NTC * 64
_ITERS = (_NTC + _NW - 1) // _NW
_BLK = 2048


def _mesh():
    return plsc.VectorSubcoreMesh(core_axis_name="c", subcore_axis_name="s")


def _sc_format(tableT, tail):
    @functools.partial(
        pl.kernel,
        mesh=_mesh(),
        out_type=jax.ShapeDtypeStruct((_LINR, 128), jnp.float32),
        scratch_types=[
            pltpu.VMEM((_D, 129), jnp.float32),
            pltpu.VMEM((_D, 128), jnp.float32),
        ],
        compiler_params=pltpu.CompilerParams(needs_layout_passes=False),
    )
    def k(tT_hbm, tail_hbm, lin_hbm, bin_, bout):
        w = lax.axis_index("s") * _NC + lax.axis_index("c")
        rowv = [lax.iota(jnp.int32, 16) + 16 * cb for cb in range(4)]

        def body(i, carry):
            tc = w + _NW * i

            @pl.when(tc < _NTC - 1)
            def _load_full():
                off = pl.multiple_of(tc * 128, 128)
                pltpu.sync_copy(tT_hbm.at[:, pl.ds(off, 128)], bin_.at[:, pl.ds(0, 128)])

            @pl.when(tc == _NTC - 1)
            def _load_part():
                pltpu.sync_copy(tail_hbm, bin_.at[:, pl.ds(0, 128)])

            @pl.when(tc < _NTC)
            def _work():
                @plsc.parallel_loop(0, 64, step=1, unroll=8)
                def _t(j):
                    for a in range(2):
                        colv = lax.broadcast(2 * j + a, (16,))
                        for cb in range(4):
                            vals = plsc.load_gather(bin_, [rowv[cb], colv])
                            bout[j, pl.ds(64 * a + 16 * cb, 16)] = vals

                pltpu.sync_copy(bout, lin_hbm.at[pl.ds(tc * 64, 64), :])

            return carry

        lax.fori_loop(0, _ITERS, body, 0)

    return k(tableT, tail)


def _sc_gather(lin, labels):
    @functools.partial(
        pl.kernel,
        mesh=_mesh(),
        out_type=jax.ShapeDtypeStruct((_B, 128), jnp.float32),
        scratch_types=[
            pltpu.VMEM((_BPW,), jnp.int32),
            pltpu.VMEM((_BPW,), jnp.int32),
            pltpu.VMEM((_BPW, 128), jnp.float32),
            pltpu.SemaphoreType.DMA,
        ],
        compiler_params=pltpu.CompilerParams(needs_layout_passes=False),
    )
    def k(lin_hbm, lab_hbm, out_hbm, lab_v, pidx_v, rows_v, sem):
        w = lax.axis_index("s") * _NC + lax.axis_index("c")
        base = w * _BPW
        pltpu.sync_copy(lab_hbm.at[pl.ds(base, _BPW)], lab_v)
        for t in range(_BPW // 16):
            pidx_v[pl.ds(16 * t, 16)] = lab_v[pl.ds(16 * t, 16)] >> 1
        copies = []
        for j in range(_NCH):
            copies.append(
                pltpu.async_copy(
                    lin_hbm.at[pidx_v.at[pl.ds(j * _CHUNK, _CHUNK)]],
                    rows_v.at[pl.ds(j * _CHUNK, _CHUNK)],
                    sem,
                )
            )
        for c in copies:
            c.wait()
        pltpu.sync_copy(rows_v, out_hbm.at[pl.ds(base, _BPW)])

    return k(lin, labels)


def _mlp_body(x_ref, lab_ref, w1_ref, b1_ref, w2_ref, b2_ref, o_ref):
    x = x_ref[...]
    par = lab_ref[...].reshape(_BLK, 1) & 1
    e = jnp.where(par == 1, x[:, _D:2 * _D], x[:, 0:_D])
    h = jnp.dot(e, w1_ref[...], preferred_element_type=jnp.float32)
    h = h + b1_ref[...]
    h = h * jax.nn.sigmoid(h)
    o = jnp.dot(h, w2_ref[...], preferred_element_type=jnp.float32)
    o_ref[...] = (o + b2_ref[...]).T


def _tc_mlp(emb128, labels3, W1, b1, W2, b2):
    grid = (_B // _BLK,)
    return pl.pallas_call(
        _mlp_body,
        grid=grid,
        in_specs=[
            pl.BlockSpec((_BLK, 128), lambda i: (i, 0)),
            pl.BlockSpec((1, 1, _BLK), lambda i: (i, 0, 0)),
            pl.BlockSpec((_D, _H), lambda i: (0, 0)),
            pl.BlockSpec((1, _H), lambda i: (0, 0)),
            pl.BlockSpec((_H, _D), lambda i: (0, 0)),
            pl.BlockSpec((1, _D), lambda i: (0, 0)),
        ],
        out_specs=pl.BlockSpec((_D, _BLK), lambda i: (0, i)),
        out_shape=jax.ShapeDtypeStruct((_D, _B), jnp.float32),
    )(emb128, labels3, W1, b1, W2, b2)


def kernel(class_labels, table, W1, b1, W2, b2):
    labels = class_labels.astype(jnp.int32)
    tableT = table.T
    tail = jnp.pad(tableT[:, (_NTC - 1) * 128:], ((0, 0), (0, 128 - _LASTW)))
    lin = _sc_format(tableT, tail)
    emb128 = _sc_gather(lin, labels)
    labels3 = labels.reshape(_B // _BLK, 1, _BLK)
    outT = _tc_mlp(emb128, labels3, W1, b1.reshape(1, _H), W2, b2.reshape(1, _D))
    return outT.T


# speedup vs baseline: 2.8913x; 1.9619x over previous
import functools

import jax
import jax.numpy as jnp
from jax import lax
from jax.experimental import pallas as pl
from jax.experimental.pallas import tpu as pltpu
from jax.experimental.pallas import tpu_sc as plsc

_B = 16384
_D = 64
_H = 256
_V = 100001
_NC = 2
_NS = 16
_NW = _NC * _NS
_BPW = _B // _NW
_CHUNK = 128
_NCH = _BPW // _CHUNK
_BLK = 2048


def _mesh():
    return plsc.VectorSubcoreMesh(core_axis_name="c", subcore_axis_name="s")


_FBLK = 2048
_FGRID = (_V + _FBLK - 1) // _FBLK
_LINR = _FGRID * _FBLK


def _fmt_body(x_ref, o_ref):
    x = x_ref[...]
    o_ref[...] = jnp.concatenate(
        [x.T, jnp.zeros((_FBLK, _D), jnp.float32)], axis=1
    )


def _tc_format(tableT):
    return pl.pallas_call(
        _fmt_body,
        grid=(_FGRID,),
        in_specs=[pl.BlockSpec((_D, _FBLK), lambda i: (0, i))],
        out_specs=pl.BlockSpec((_FBLK, 128), lambda i: (i, 0)),
        out_shape=jax.ShapeDtypeStruct((_LINR, 128), jnp.float32),
    )(tableT)


def _sc_gather(lin, labels):
    @functools.partial(
        pl.kernel,
        mesh=_mesh(),
        out_type=jax.ShapeDtypeStruct((_B, 128), jnp.float32),
        scratch_types=[
            pltpu.VMEM((_BPW,), jnp.int32),
            pltpu.VMEM((_BPW, 128), jnp.float32),
            pltpu.SemaphoreType.DMA,
        ],
        compiler_params=pltpu.CompilerParams(needs_layout_passes=False),
    )
    def k(lin_hbm, lab_hbm, out_hbm, lab_v, rows_v, sem):
        w = lax.axis_index("s") * _NC + lax.axis_index("c")
        base = w * _BPW
        pltpu.sync_copy(lab_hbm.at[pl.ds(base, _BPW)], lab_v)
        copies = []
        for j in range(_NCH):
            copies.append(
                pltpu.async_copy(
                    lin_hbm.at[lab_v.at[pl.ds(j * _CHUNK, _CHUNK)]],
                    rows_v.at[pl.ds(j * _CHUNK, _CHUNK)],
                    sem,
                )
            )
        for c in copies:
            c.wait()
        pltpu.sync_copy(rows_v, out_hbm.at[pl.ds(base, _BPW)])

    return k(lin, labels)


def _mlp_body(x_ref, w1_ref, b1_ref, w2_ref, b2_ref, o_ref):
    x = x_ref[...]
    e = x[:, 0:_D]
    h = jnp.dot(e, w1_ref[...], preferred_element_type=jnp.float32)
    h = h + b1_ref[...]
    h = h * jax.nn.sigmoid(h)
    o = jnp.dot(h, w2_ref[...], preferred_element_type=jnp.float32)
    o_ref[...] = (o + b2_ref[...]).T


def _tc_mlp(emb128, W1, b1, W2, b2):
    grid = (_B // _BLK,)
    return pl.pallas_call(
        _mlp_body,
        grid=grid,
        in_specs=[
            pl.BlockSpec((_BLK, 128), lambda i: (i, 0)),
            pl.BlockSpec((_D, _H), lambda i: (0, 0)),
            pl.BlockSpec((1, _H), lambda i: (0, 0)),
            pl.BlockSpec((_H, _D), lambda i: (0, 0)),
            pl.BlockSpec((1, _D), lambda i: (0, 0)),
        ],
        out_specs=pl.BlockSpec((_D, _BLK), lambda i: (0, i)),
        out_shape=jax.ShapeDtypeStruct((_D, _B), jnp.float32),
    )(emb128, W1, b1, W2, b2)


def kernel(class_labels, table, W1, b1, W2, b2):
    labels = class_labels.astype(jnp.int32)
    lin = _tc_format(table.T)
    emb128 = _sc_gather(lin, labels)
    outT = _tc_mlp(emb128, W1, b1.reshape(1, _H), W2, b2.reshape(1, _D))
    return outT.T
